# baseline (speedup 1.0000x reference)
import functools

import jax
import jax.numpy as jnp
from jax import lax
from jax.experimental import pallas as pl
from jax.experimental.pallas import tpu as pltpu
from jax.experimental.pallas import tpu_sc as plsc

EPS = 1e-06

NC = 2
NS = 16
L = 16
NW = NC * NS
BLK = 128


@functools.lru_cache(maxsize=None)
def _build(B: int, T: int):
    nblk = B // BLK
    assert B % (NW * BLK) == 0
    bpw = nblk // NW
    cb = 8
    nchunk = bpw // cb
    assert nchunk % 2 == 0 and nchunk >= 4

    mesh = plsc.VectorSubcoreMesh(
        core_axis_name="c", subcore_axis_name="s", num_cores=NC, num_subcores=NS
    )

    buf = lambda shape, dt: pltpu.VMEM(shape, dt)

    @functools.partial(
        pl.kernel,
        out_type=(
            jax.ShapeDtypeStruct((nblk, 2, BLK), jnp.float32),
            jax.ShapeDtypeStruct((B,), jnp.float32),
        ),
        mesh=mesh,
        compiler_params=pltpu.CompilerParams(needs_layout_passes=False),
        scratch_types=[
            pltpu.VMEM((T * 9,), jnp.float32),
            buf((cb * BLK,), jnp.int32), buf((cb * BLK,), jnp.int32),
            buf((cb, 2, BLK), jnp.float32), buf((cb, 2, BLK), jnp.float32),
            buf((cb, 2, BLK), jnp.float32), buf((cb, 2, BLK), jnp.float32),
            buf((cb * BLK,), jnp.float32), buf((cb * BLK,), jnp.float32),
            pltpu.SemaphoreType.DMA, pltpu.SemaphoreType.DMA,
            pltpu.SemaphoreType.DMA, pltpu.SemaphoreType.DMA,
        ],
    )
    def homog(h_hbm, xy_hbm, t_hbm, oxy_hbm, ow_hbm, h_v,
              t_v0, t_v1, xy_v0, xy_v1, oxy_v0, oxy_v1, ow_v0, ow_v1,
              in_sem0, in_sem1, out_sem0, out_sem1):
        t_v = (t_v0, t_v1)
        xy_v = (xy_v0, xy_v1)
        oxy_v = (oxy_v0, oxy_v1)
        ow_v = (ow_v0, ow_v1)
        in_sem = (in_sem0, in_sem1)
        out_sem = (out_sem0, out_sem1)

        wid = lax.axis_index("s") * NC + lax.axis_index("c")
        base = wid * bpw

        def start_in(ci, p):
            cbase = base + ci * cb
            pltpu.async_copy(t_hbm.at[pl.ds(cbase * BLK, cb * BLK)], t_v[p], in_sem[p])
            pltpu.async_copy(xy_hbm.at[pl.ds(cbase, cb)], xy_v[p], in_sem[p])

        def wait_in(ci, p):
            cbase = base + ci * cb
            pltpu.make_async_copy(t_hbm.at[pl.ds(cbase * BLK, cb * BLK)], t_v[p], in_sem[p]).wait()
            pltpu.make_async_copy(xy_hbm.at[pl.ds(cbase, cb)], xy_v[p], in_sem[p]).wait()

        def start_out(ci, p):
            cbase = base + ci * cb
            pltpu.async_copy(oxy_v[p], oxy_hbm.at[pl.ds(cbase, cb)], out_sem[p])
            pltpu.async_copy(ow_v[p], ow_hbm.at[pl.ds(cbase * BLK, cb * BLK)], out_sem[p])

        def wait_out(ci, p):
            cbase = base + ci * cb
            pltpu.make_async_copy(oxy_v[p], oxy_hbm.at[pl.ds(cbase, cb)], out_sem[p]).wait()
            pltpu.make_async_copy(ow_v[p], ow_hbm.at[pl.ds(cbase * BLK, cb * BLK)], out_sem[p]).wait()

        h_plane = [h_v.at[pl.ds(k * T, T)] for k in range(9)]

        def compute(p):
            @plsc.parallel_loop(0, cb, step=1, unroll=1)
            def blk_body(b):
                for s in range(BLK // L):
                    x = xy_v[p][b, 0, pl.ds(s * L, L)]
                    y = xy_v[p][b, 1, pl.ds(s * L, L)]
                    tvec = t_v[p][pl.ds(b * BLK + s * L, L)]
                    h00 = plsc.load_gather(h_plane[0], [tvec])
                    h01 = plsc.load_gather(h_plane[1], [tvec])
                    h02 = plsc.load_gather(h_plane[2], [tvec])
                    h10 = plsc.load_gather(h_plane[3], [tvec])
                    h11 = plsc.load_gather(h_plane[4], [tvec])
                    h12 = plsc.load_gather(h_plane[5], [tvec])
                    h20 = plsc.load_gather(h_plane[6], [tvec])
                    h21 = plsc.load_gather(h_plane[7], [tvec])
                    h22 = plsc.load_gather(h_plane[8], [tvec])
                    o0 = x * h00 + y * h01 + h02
                    o1 = x * h10 + y * h11 + h12
                    w = x * h20 + y * h21 + h22
                    wb = plsc.bitcast(w, jnp.int32)
                    mag = plsc.bitcast(jnp.maximum(jnp.abs(w), EPS), jnp.int32)
                    db = (wb & jnp.int32(-2147483648)) | mag
                    den = jnp.where(w == 0.0, 0.0, plsc.bitcast(db, jnp.float32))
                    oxy_v[p][b, 0, pl.ds(s * L, L)] = o0 / den
                    oxy_v[p][b, 1, pl.ds(s * L, L)] = o1 / den
                    ow_v[p][pl.ds(b * BLK + s * L, L)] = w

        start_in(0, 0)
        pltpu.sync_copy(h_hbm, h_v)

        np_ = nchunk // 2

        def pair_body(i, carry):
            for p in (0, 1):
                ci = 2 * i + p
                nxt = ci + 1

                @pl.when(nxt < nchunk)
                def _():
                    start_in(nxt, 1 - p)

                wait_in(ci, p)

                @pl.when(ci >= 2)
                def _():
                    wait_out(ci - 2, p)

                compute(p)
                start_out(ci, p)
            return carry

        lax.fori_loop(0, np_, pair_body, 0)
        wait_out(nchunk - 2, 0)
        wait_out(nchunk - 1, 1)

    return homog


def kernel(xy, t, H):
    B = xy.shape[0]
    T = H.shape[0]
    homog = _build(B, T)
    h_planes = H.transpose(1, 2, 0).reshape(-1)
    xy_b = xy.reshape(B // BLK, BLK, 2).transpose(0, 2, 1)
    oxy, ow = homog(h_planes, xy_b, t)
    xy_t = oxy.transpose(0, 2, 1).reshape(B, 2)
    return xy_t, ow.reshape(B, 1)


# speedup vs baseline: 1.0862x; 1.0862x over previous
import functools

import jax
import jax.numpy as jnp
from jax import lax
from jax.experimental import pallas as pl
from jax.experimental.pallas import tpu as pltpu
from jax.experimental.pallas import tpu_sc as plsc

EPS = 1e-06

NC = 2
NS = 16
L = 16
NW = NC * NS
BLK = 128


@functools.lru_cache(maxsize=None)
def _build(B: int, T: int):
    nblk = B // BLK
    assert B % (NW * BLK) == 0
    bpw = nblk // NW
    cb = 16
    nchunk = bpw // cb
    assert nchunk % 2 == 0 and nchunk >= 4

    mesh = plsc.VectorSubcoreMesh(
        core_axis_name="c", subcore_axis_name="s", num_cores=NC, num_subcores=NS
    )

    buf = lambda shape, dt: pltpu.VMEM(shape, dt)

    @functools.partial(
        pl.kernel,
        out_type=(
            jax.ShapeDtypeStruct((nblk, 2, BLK), jnp.float32),
            jax.ShapeDtypeStruct((B,), jnp.float32),
        ),
        mesh=mesh,
        compiler_params=pltpu.CompilerParams(needs_layout_passes=False),
        scratch_types=[
            pltpu.VMEM((T * 9,), jnp.float32),
            buf((cb * BLK,), jnp.int32), buf((cb * BLK,), jnp.int32),
            buf((cb, 2, BLK), jnp.float32), buf((cb, 2, BLK), jnp.float32),
            buf((cb, 2, BLK), jnp.float32), buf((cb, 2, BLK), jnp.float32),
            buf((cb * BLK,), jnp.float32), buf((cb * BLK,), jnp.float32),
            pltpu.SemaphoreType.DMA, pltpu.SemaphoreType.DMA,
            pltpu.SemaphoreType.DMA, pltpu.SemaphoreType.DMA,
        ],
    )
    def homog(h_hbm, xy_hbm, t_hbm, oxy_hbm, ow_hbm, h_v,
              t_v0, t_v1, xy_v0, xy_v1, oxy_v0, oxy_v1, ow_v0, ow_v1,
              in_sem0, in_sem1, out_sem0, out_sem1):
        t_v = (t_v0, t_v1)
        xy_v = (xy_v0, xy_v1)
        oxy_v = (oxy_v0, oxy_v1)
        ow_v = (ow_v0, ow_v1)
        in_sem = (in_sem0, in_sem1)
        out_sem = (out_sem0, out_sem1)

        wid = lax.axis_index("s") * NC + lax.axis_index("c")
        base = wid * bpw

        def start_in(ci, p):
            cbase = base + ci * cb
            pltpu.async_copy(t_hbm.at[pl.ds(cbase * BLK, cb * BLK)], t_v[p], in_sem[p])
            pltpu.async_copy(xy_hbm.at[pl.ds(cbase, cb)], xy_v[p], in_sem[p])

        def wait_in(ci, p):
            cbase = base + ci * cb
            pltpu.make_async_copy(t_hbm.at[pl.ds(cbase * BLK, cb * BLK)], t_v[p], in_sem[p]).wait()
            pltpu.make_async_copy(xy_hbm.at[pl.ds(cbase, cb)], xy_v[p], in_sem[p]).wait()

        def start_out(ci, p):
            cbase = base + ci * cb
            pltpu.async_copy(oxy_v[p], oxy_hbm.at[pl.ds(cbase, cb)], out_sem[p])
            pltpu.async_copy(ow_v[p], ow_hbm.at[pl.ds(cbase * BLK, cb * BLK)], out_sem[p])

        def wait_out(ci, p):
            cbase = base + ci * cb
            pltpu.make_async_copy(oxy_v[p], oxy_hbm.at[pl.ds(cbase, cb)], out_sem[p]).wait()
            pltpu.make_async_copy(ow_v[p], ow_hbm.at[pl.ds(cbase * BLK, cb * BLK)], out_sem[p]).wait()

        h_plane = [h_v.at[pl.ds(k * T, T)] for k in range(9)]

        def compute(p):
            @plsc.parallel_loop(0, cb, step=1, unroll=1)
            def blk_body(b):
                for s in range(BLK // L):
                    x = xy_v[p][b, 0, pl.ds(s * L, L)]
                    y = xy_v[p][b, 1, pl.ds(s * L, L)]
                    tvec = t_v[p][pl.ds(b * BLK + s * L, L)]
                    h00 = plsc.load_gather(h_plane[0], [tvec])
                    h01 = plsc.load_gather(h_plane[1], [tvec])
                    h02 = plsc.load_gather(h_plane[2], [tvec])
                    h10 = plsc.load_gather(h_plane[3], [tvec])
                    h11 = plsc.load_gather(h_plane[4], [tvec])
                    h12 = plsc.load_gather(h_plane[5], [tvec])
                    h20 = plsc.load_gather(h_plane[6], [tvec])
                    h21 = plsc.load_gather(h_plane[7], [tvec])
                    h22 = plsc.load_gather(h_plane[8], [tvec])
                    o0 = x * h00 + y * h01 + h02
                    o1 = x * h10 + y * h11 + h12
                    w = x * h20 + y * h21 + h22
                    wb = plsc.bitcast(w, jnp.int32)
                    mag = plsc.bitcast(jnp.maximum(jnp.abs(w), EPS), jnp.int32)
                    db = (wb & jnp.int32(-2147483648)) | mag
                    den = jnp.where(w == 0.0, 0.0, plsc.bitcast(db, jnp.float32))
                    oxy_v[p][b, 0, pl.ds(s * L, L)] = o0 / den
                    oxy_v[p][b, 1, pl.ds(s * L, L)] = o1 / den
                    ow_v[p][pl.ds(b * BLK + s * L, L)] = w

        start_in(0, 0)
        pltpu.sync_copy(h_hbm, h_v)

        np_ = nchunk // 2

        def pair_body(i, carry):
            for p in (0, 1):
                ci = 2 * i + p
                nxt = ci + 1

                @pl.when(nxt < nchunk)
                def _():
                    start_in(nxt, 1 - p)

                wait_in(ci, p)

                @pl.when(ci >= 2)
                def _():
                    wait_out(ci - 2, p)

                compute(p)
                start_out(ci, p)
            return carry

        lax.fori_loop(0, np_, pair_body, 0)
        wait_out(nchunk - 2, 0)
        wait_out(nchunk - 1, 1)

    return homog


def kernel(xy, t, H):
    B = xy.shape[0]
    T = H.shape[0]
    homog = _build(B, T)
    h_planes = H.transpose(1, 2, 0).reshape(-1)
    xy_b = xy.reshape(B // BLK, BLK, 2).transpose(0, 2, 1)
    oxy, ow = homog(h_planes, xy_b, t)
    xy_t = oxy.transpose(0, 2, 1).reshape(B, 2)
    return xy_t, ow.reshape(B, 1)
